# baseline (speedup 1.0000x reference)
import functools

import jax
import jax.numpy as jnp
from jax import lax
from jax.experimental import pallas as pl
from jax.experimental.pallas import tpu as pltpu
from jax.experimental.pallas import tpu_sc as plsc

N = 10000
E = 320000
N_EXT = 10240
CH = 128
NSUB = 16
NCORE = 2
CHUNKS = 2560
E_PAD = CHUNKS * CH
ROWS_PER_SUB = N_EXT // NSUB
BLK = 512


def _sc_mesh():
    return plsc.VectorSubcoreMesh(core_axis_name="c", subcore_axis_name="s")


def _chunk_loop(h_hbm, src_hbm, dst_hbm, acc, base, cps, blk_b,
                srcb0, dstb0, srcb1, dstb1, rows0, rows1,
                semg0, semg1, semi0, semi1, semsc0, semsc1, after_prime):
    nb_total = cps // blk_b
    bufs = [(srcb0, dstb0), (srcb1, dstb1)]

    def wait_gather(buf, sem):
        pltpu.make_async_copy(h_hbm.at[srcb0.at[0]], buf, sem).wait()

    def wait_scatter(buf, sem):
        pltpu.make_async_copy(buf, acc.at[dstb0.at[0]], sem).wait()

    pltpu.sync_copy(src_hbm.at[pl.ds(base, blk_b)], srcb0)
    pltpu.sync_copy(dst_hbm.at[pl.ds(base, blk_b)], dstb0)
    pltpu.async_copy(h_hbm.at[srcb0.at[0]], rows0, semg0)
    pltpu.async_copy(h_hbm.at[srcb0.at[1]], rows1, semg1)
    after_prime()

    for nb in range(nb_total):
        srcb, dstb = bufs[nb % 2]
        nsrcb, ndstb = bufs[(nb + 1) % 2]
        have_next = nb + 1 < nb_total
        if have_next:
            nbase = base + (nb + 1) * blk_b
            ic_s = pltpu.async_copy(src_hbm.at[pl.ds(nbase, blk_b)],
                                    nsrcb, semi0)
            ic_d = pltpu.async_copy(dst_hbm.at[pl.ds(nbase, blk_b)],
                                    ndstb, semi1)

        @pl.loop(0, blk_b // 2 - 1)
        def _(p):
            j = 2 * p
            wait_gather(rows0, semg0)
            pltpu.async_copy(rows0, acc.at[dstb.at[j]], semsc0, add=True)
            wait_gather(rows1, semg1)
            pltpu.async_copy(rows1, acc.at[dstb.at[j + 1]], semsc1, add=True)
            wait_scatter(rows0, semsc0)
            pltpu.async_copy(h_hbm.at[srcb.at[j + 2]], rows0, semg0)
            wait_scatter(rows1, semsc1)
            pltpu.async_copy(h_hbm.at[srcb.at[j + 3]], rows1, semg1)

        wait_gather(rows0, semg0)
        pltpu.async_copy(rows0, acc.at[dstb.at[blk_b - 2]], semsc0, add=True)
        wait_gather(rows1, semg1)
        pltpu.async_copy(rows1, acc.at[dstb.at[blk_b - 1]], semsc1, add=True)
        wait_scatter(rows0, semsc0)
        if have_next:
            ic_s.wait()
            ic_d.wait()
            pltpu.async_copy(h_hbm.at[nsrcb.at[0]], rows0, semg0)
        wait_scatter(rows1, semsc1)
        if have_next:
            pltpu.async_copy(h_hbm.at[nsrcb.at[1]], rows1, semg1)


def _make_agg(split):
    fh = 128
    cps = CHUNKS // (NCORE * NSUB) if not split else CHUNKS // NSUB
    blk_b = 16
    out_t = (jax.ShapeDtypeStruct((N_EXT, fh), jnp.float32),
             jax.ShapeDtypeStruct((N_EXT, fh), jnp.float32))

    @functools.partial(
        pl.kernel, mesh=_sc_mesh(), out_type=out_t,
        scratch_types=[
            pltpu.VMEM((blk_b, CH), jnp.int32),
            pltpu.VMEM((blk_b, CH), jnp.int32),
            pltpu.VMEM((blk_b, CH), jnp.int32),
            pltpu.VMEM((blk_b, CH), jnp.int32),
            pltpu.VMEM((CH, fh), jnp.float32),
            pltpu.VMEM((CH, fh), jnp.float32),
            pltpu.VMEM_SHARED((N_EXT, fh), jnp.float32),
            pltpu.SemaphoreType.DMA,
            pltpu.SemaphoreType.DMA,
            pltpu.SemaphoreType.DMA,
            pltpu.SemaphoreType.DMA,
            pltpu.SemaphoreType.DMA,
            pltpu.SemaphoreType.DMA,
            pltpu.SemaphoreType.DMA,
        ],
    )
    def agg(ha_hbm, hb_hbm, src_hbm, dst_hbm, zeros_hbm, o0, o1,
            srcb0, dstb0, srcb1, dstb1, rows0, rows1, acc,
            semg0, semg1, semi0, semi1, semsc0, semsc1, semz):
        c = lax.axis_index("c")
        s = lax.axis_index("s")
        r0 = s * ROWS_PER_SUB
        zc = pltpu.async_copy(zeros_hbm.at[pl.ds(r0, ROWS_PER_SUB)],
                              acc.at[pl.ds(r0, ROWS_PER_SUB)], semz)

        def after_prime():
            zc.wait()
            plsc.subcore_barrier()

        if not split:
            base = (c * NSUB + s) * cps
        else:
            base = s * cps

        @pl.when(c == 0)
        def _():
            _chunk_loop(ha_hbm, src_hbm, dst_hbm, acc, base, cps, blk_b,
                        srcb0, dstb0, srcb1, dstb1, rows0, rows1,
                        semg0, semg1, semi0, semi1, semsc0, semsc1,
                        after_prime)

        @pl.when(c == 1)
        def _():
            _chunk_loop(hb_hbm, src_hbm, dst_hbm, acc, base, cps, blk_b,
                        srcb0, dstb0, srcb1, dstb1, rows0, rows1,
                        semg0, semg1, semi0, semi1, semsc0, semsc1,
                        after_prime)

        plsc.subcore_barrier()

        @pl.when(c == 0)
        def _():
            pltpu.sync_copy(acc.at[pl.ds(r0, ROWS_PER_SUB)],
                            o0.at[pl.ds(r0, ROWS_PER_SUB)])

        @pl.when(c == 1)
        def _():
            pltpu.sync_copy(acc.at[pl.ds(r0, ROWS_PER_SUB)],
                            o1.at[pl.ds(r0, ROWS_PER_SUB)])

    return agg


def _row_mask(i, y):
    rows = i * BLK + lax.broadcasted_iota(jnp.int32, (BLK, 1), 0)
    return jnp.where(rows < N, y, 0.0)


def _make_mlp1():
    def body(h, m0, m1, w, b, oa, ob):
        i = pl.program_id(0)
        x = h[...] + m0[...] + m1[...]
        y = jnp.dot(x, w[...], preferred_element_type=jnp.float32) + b[...]
        y = _row_mask(i, jnp.maximum(y, 0.0))
        oa[...] = y[:, :128]
        ob[...] = y[:, 128:]

    blk_in = pl.BlockSpec((BLK, 128), lambda i: (i, 0))
    return pl.pallas_call(
        body,
        grid=(N_EXT // BLK,),
        in_specs=[blk_in, blk_in, blk_in,
                  pl.BlockSpec((128, 256), lambda i: (0, 0)),
                  pl.BlockSpec((1, 256), lambda i: (0, 0))],
        out_specs=[pl.BlockSpec((BLK, 128), lambda i: (i, 0)),
                   pl.BlockSpec((BLK, 128), lambda i: (i, 0))],
        out_shape=(jax.ShapeDtypeStruct((N_EXT, 128), jnp.float32),
                   jax.ShapeDtypeStruct((N_EXT, 128), jnp.float32)),
    )


def _make_mlp_split(f_out, split_out):
    def body(ha, hb, ma, mb, w, b, *outs):
        i = pl.program_id(0)
        x = jnp.concatenate([ha[...] + ma[...], hb[...] + mb[...]], axis=1)
        y = jnp.dot(x, w[...], preferred_element_type=jnp.float32) + b[...]
        y = _row_mask(i, jnp.maximum(y, 0.0))
        if split_out:
            outs[0][...] = y[:, :f_out // 2]
            outs[1][...] = y[:, f_out // 2:]
        else:
            outs[0][...] = y

    blk_in = pl.BlockSpec((BLK, 128), lambda i: (i, 0))
    if split_out:
        out_specs = [pl.BlockSpec((BLK, f_out // 2), lambda i: (i, 0)),
                     pl.BlockSpec((BLK, f_out // 2), lambda i: (i, 0))]
        out_shape = (jax.ShapeDtypeStruct((N_EXT, f_out // 2), jnp.float32),
                     jax.ShapeDtypeStruct((N_EXT, f_out // 2), jnp.float32))
    else:
        out_specs = [pl.BlockSpec((BLK, f_out), lambda i: (i, 0))]
        out_shape = (jax.ShapeDtypeStruct((N_EXT, f_out), jnp.float32),)

    return pl.pallas_call(
        body,
        grid=(N_EXT // BLK,),
        in_specs=[blk_in, blk_in, blk_in, blk_in,
                  pl.BlockSpec((256, f_out), lambda i: (0, 0)),
                  pl.BlockSpec((1, f_out), lambda i: (0, 0))],
        out_specs=out_specs,
        out_shape=out_shape,
    )


def kernel(x, edge_index, W1, b1, W2, b2, W3, b3):
    src = edge_index[0].astype(jnp.int32)
    dst = edge_index[1].astype(jnp.int32)
    npad = E_PAD - E
    pad_ids = (jnp.arange(npad, dtype=jnp.int32) % 8) + N
    srcm = jnp.concatenate([src, pad_ids]).reshape(CHUNKS, CH)
    dstm = jnp.concatenate([dst, pad_ids]).reshape(CHUNKS, CH)
    h0 = jnp.zeros((N_EXT, 128), jnp.float32).at[:N].set(x)
    z128 = jnp.zeros((N_EXT, 128), jnp.float32)

    agg_full = _make_agg(split=False)
    agg_split = _make_agg(split=True)
    mlp1 = _make_mlp1()
    mlp2 = _make_mlp_split(256, split_out=True)
    mlp3 = _make_mlp_split(128, split_out=False)

    m0, m1 = agg_full(h0, h0, srcm, dstm, z128)
    ha, hb = mlp1(h0, m0, m1, W1, b1.reshape(1, -1))
    ma, mb = agg_split(ha, hb, srcm, dstm, z128)
    ha, hb = mlp2(ha, hb, ma, mb, W2, b2.reshape(1, -1))
    ma, mb = agg_split(ha, hb, srcm, dstm, z128)
    (h3,) = mlp3(ha, hb, ma, mb, W3, b3.reshape(1, -1))
    return h3[:N]


# speedup vs baseline: 1.2108x; 1.2108x over previous
import functools

import jax
import jax.numpy as jnp
from jax import lax
from jax.experimental import pallas as pl
from jax.experimental.pallas import tpu as pltpu
from jax.experimental.pallas import tpu_sc as plsc

N = 10000
E = 320000
N_EXT = 10240
CH = 128
NSUB = 16
NCORE = 2
CHUNKS = 2560
E_PAD = CHUNKS * CH
ROWS_PER_SUB = N_EXT // NSUB
BLK = 512


def _sc_mesh():
    return plsc.VectorSubcoreMesh(core_axis_name="c", subcore_axis_name="s")


def _chunk_loop(h_hbm, src_hbm, dst_hbm, acc, base, cps, blk_b,
                srcb0, dstb0, srcb1, dstb1, rows0, rows1,
                semg0, semg1, semi0, semi1, semsc0, semsc1, after_prime):
    nb_total = cps // blk_b
    bufs = [(srcb0, dstb0), (srcb1, dstb1)]

    def wait_gather(buf, sem):
        pltpu.make_async_copy(h_hbm.at[srcb0.at[0]], buf, sem).wait()

    def wait_scatter(buf, sem):
        pltpu.make_async_copy(buf, acc.at[dstb0.at[0]], sem).wait()

    pltpu.sync_copy(src_hbm.at[pl.ds(base, blk_b)], srcb0)
    pltpu.sync_copy(dst_hbm.at[pl.ds(base, blk_b)], dstb0)
    pltpu.async_copy(h_hbm.at[srcb0.at[0]], rows0, semg0)
    pltpu.async_copy(h_hbm.at[srcb0.at[1]], rows1, semg1)
    after_prime()

    for nb in range(nb_total):
        srcb, dstb = bufs[nb % 2]
        nsrcb, ndstb = bufs[(nb + 1) % 2]
        have_next = nb + 1 < nb_total
        if have_next:
            nbase = base + (nb + 1) * blk_b
            ic_s = pltpu.async_copy(src_hbm.at[pl.ds(nbase, blk_b)],
                                    nsrcb, semi0)
            ic_d = pltpu.async_copy(dst_hbm.at[pl.ds(nbase, blk_b)],
                                    ndstb, semi1)

        @pl.loop(0, blk_b // 2 - 1)
        def _(p):
            j = 2 * p
            wait_gather(rows0, semg0)
            pltpu.sync_copy(rows0, acc.at[dstb.at[j]], add=True)
            pltpu.async_copy(h_hbm.at[srcb.at[j + 2]], rows0, semg0)
            wait_gather(rows1, semg1)
            pltpu.sync_copy(rows1, acc.at[dstb.at[j + 1]], add=True)
            pltpu.async_copy(h_hbm.at[srcb.at[j + 3]], rows1, semg1)

        wait_gather(rows0, semg0)
        pltpu.sync_copy(rows0, acc.at[dstb.at[blk_b - 2]], add=True)
        if have_next:
            ic_s.wait()
            ic_d.wait()
            pltpu.async_copy(h_hbm.at[nsrcb.at[0]], rows0, semg0)
        wait_gather(rows1, semg1)
        pltpu.sync_copy(rows1, acc.at[dstb.at[blk_b - 1]], add=True)
        if have_next:
            pltpu.async_copy(h_hbm.at[nsrcb.at[1]], rows1, semg1)


def _make_agg(split):
    fh = 128
    cps = CHUNKS // (NCORE * NSUB) if not split else CHUNKS // NSUB
    blk_b = 16
    out_t = (jax.ShapeDtypeStruct((N_EXT, fh), jnp.float32),
             jax.ShapeDtypeStruct((N_EXT, fh), jnp.float32))

    @functools.partial(
        pl.kernel, mesh=_sc_mesh(), out_type=out_t,
        scratch_types=[
            pltpu.VMEM((blk_b, CH), jnp.int32),
            pltpu.VMEM((blk_b, CH), jnp.int32),
            pltpu.VMEM((blk_b, CH), jnp.int32),
            pltpu.VMEM((blk_b, CH), jnp.int32),
            pltpu.VMEM((CH, fh), jnp.float32),
            pltpu.VMEM((CH, fh), jnp.float32),
            pltpu.VMEM_SHARED((N_EXT, fh), jnp.float32),
            pltpu.SemaphoreType.DMA,
            pltpu.SemaphoreType.DMA,
            pltpu.SemaphoreType.DMA,
            pltpu.SemaphoreType.DMA,
            pltpu.SemaphoreType.DMA,
            pltpu.SemaphoreType.DMA,
            pltpu.SemaphoreType.DMA,
        ],
    )
    def agg(ha_hbm, hb_hbm, src_hbm, dst_hbm, zeros_hbm, o0, o1,
            srcb0, dstb0, srcb1, dstb1, rows0, rows1, acc,
            semg0, semg1, semi0, semi1, semsc0, semsc1, semz):
        c = lax.axis_index("c")
        s = lax.axis_index("s")
        r0 = s * ROWS_PER_SUB
        zc = pltpu.async_copy(zeros_hbm.at[pl.ds(r0, ROWS_PER_SUB)],
                              acc.at[pl.ds(r0, ROWS_PER_SUB)], semz)

        def after_prime():
            zc.wait()
            plsc.subcore_barrier()

        if not split:
            base = (c * NSUB + s) * cps
        else:
            base = s * cps

        @pl.when(c == 0)
        def _():
            _chunk_loop(ha_hbm, src_hbm, dst_hbm, acc, base, cps, blk_b,
                        srcb0, dstb0, srcb1, dstb1, rows0, rows1,
                        semg0, semg1, semi0, semi1, semsc0, semsc1,
                        after_prime)

        @pl.when(c == 1)
        def _():
            _chunk_loop(hb_hbm, src_hbm, dst_hbm, acc, base, cps, blk_b,
                        srcb0, dstb0, srcb1, dstb1, rows0, rows1,
                        semg0, semg1, semi0, semi1, semsc0, semsc1,
                        after_prime)

        plsc.subcore_barrier()

        @pl.when(c == 0)
        def _():
            pltpu.sync_copy(acc.at[pl.ds(r0, ROWS_PER_SUB)],
                            o0.at[pl.ds(r0, ROWS_PER_SUB)])

        @pl.when(c == 1)
        def _():
            pltpu.sync_copy(acc.at[pl.ds(r0, ROWS_PER_SUB)],
                            o1.at[pl.ds(r0, ROWS_PER_SUB)])

    return agg


def _row_mask(i, y):
    rows = i * BLK + lax.broadcasted_iota(jnp.int32, (BLK, 1), 0)
    return jnp.where(rows < N, y, 0.0)


def _make_mlp1():
    def body(h, m0, m1, w, b, oa, ob):
        i = pl.program_id(0)
        x = h[...] + m0[...] + m1[...]
        y = jnp.dot(x, w[...], preferred_element_type=jnp.float32) + b[...]
        y = _row_mask(i, jnp.maximum(y, 0.0))
        oa[...] = y[:, :128]
        ob[...] = y[:, 128:]

    blk_in = pl.BlockSpec((BLK, 128), lambda i: (i, 0))
    return pl.pallas_call(
        body,
        grid=(N_EXT // BLK,),
        in_specs=[blk_in, blk_in, blk_in,
                  pl.BlockSpec((128, 256), lambda i: (0, 0)),
                  pl.BlockSpec((1, 256), lambda i: (0, 0))],
        out_specs=[pl.BlockSpec((BLK, 128), lambda i: (i, 0)),
                   pl.BlockSpec((BLK, 128), lambda i: (i, 0))],
        out_shape=(jax.ShapeDtypeStruct((N_EXT, 128), jnp.float32),
                   jax.ShapeDtypeStruct((N_EXT, 128), jnp.float32)),
    )


def _make_mlp_split(f_out, split_out):
    def body(ha, hb, ma, mb, w, b, *outs):
        i = pl.program_id(0)
        x = jnp.concatenate([ha[...] + ma[...], hb[...] + mb[...]], axis=1)
        y = jnp.dot(x, w[...], preferred_element_type=jnp.float32) + b[...]
        y = _row_mask(i, jnp.maximum(y, 0.0))
        if split_out:
            outs[0][...] = y[:, :f_out // 2]
            outs[1][...] = y[:, f_out // 2:]
        else:
            outs[0][...] = y

    blk_in = pl.BlockSpec((BLK, 128), lambda i: (i, 0))
    if split_out:
        out_specs = [pl.BlockSpec((BLK, f_out // 2), lambda i: (i, 0)),
                     pl.BlockSpec((BLK, f_out // 2), lambda i: (i, 0))]
        out_shape = (jax.ShapeDtypeStruct((N_EXT, f_out // 2), jnp.float32),
                     jax.ShapeDtypeStruct((N_EXT, f_out // 2), jnp.float32))
    else:
        out_specs = [pl.BlockSpec((BLK, f_out), lambda i: (i, 0))]
        out_shape = (jax.ShapeDtypeStruct((N_EXT, f_out), jnp.float32),)

    return pl.pallas_call(
        body,
        grid=(N_EXT // BLK,),
        in_specs=[blk_in, blk_in, blk_in, blk_in,
                  pl.BlockSpec((256, f_out), lambda i: (0, 0)),
                  pl.BlockSpec((1, f_out), lambda i: (0, 0))],
        out_specs=out_specs,
        out_shape=out_shape,
    )


def kernel(x, edge_index, W1, b1, W2, b2, W3, b3):
    src = edge_index[0].astype(jnp.int32)
    dst = edge_index[1].astype(jnp.int32)
    npad = E_PAD - E
    pad_ids = (jnp.arange(npad, dtype=jnp.int32) % 8) + N
    srcm = jnp.concatenate([src, pad_ids]).reshape(CHUNKS, CH)
    dstm = jnp.concatenate([dst, pad_ids]).reshape(CHUNKS, CH)
    h0 = jnp.zeros((N_EXT, 128), jnp.float32).at[:N].set(x)
    z128 = jnp.zeros((N_EXT, 128), jnp.float32)

    agg_full = _make_agg(split=False)
    agg_split = _make_agg(split=True)
    mlp1 = _make_mlp1()
    mlp2 = _make_mlp_split(256, split_out=True)
    mlp3 = _make_mlp_split(128, split_out=False)

    m0, m1 = agg_full(h0, h0, srcm, dstm, z128)
    ha, hb = mlp1(h0, m0, m1, W1, b1.reshape(1, -1))
    ma, mb = agg_split(ha, hb, srcm, dstm, z128)
    ha, hb = mlp2(ha, hb, ma, mb, W2, b2.reshape(1, -1))
    ma, mb = agg_split(ha, hb, srcm, dstm, z128)
    (h3,) = mlp3(ha, hb, ma, mb, W3, b3.reshape(1, -1))
    return h3[:N]


# speedup vs baseline: 1.3421x; 1.1085x over previous
import functools

import jax
import jax.numpy as jnp
from jax import lax
from jax.experimental import pallas as pl
from jax.experimental.pallas import tpu as pltpu
from jax.experimental.pallas import tpu_sc as plsc

N = 10000
E = 320000
N_EXT = 10240
CH = 128
NSUB = 16
NCORE = 2
CHUNKS = 2560
E_PAD = CHUNKS * CH
ROWS_PER_SUB = N_EXT // NSUB
BLK = 512


def _sc_mesh():
    return plsc.VectorSubcoreMesh(core_axis_name="c", subcore_axis_name="s")


def _chunk_loop(h_hbm, src_hbm, dst_hbm, acc, base, cps, blk_b,
                srcb0, dstb0, srcb1, dstb1, rows0, rows1,
                semg0, semg1, semi0, semi1, semsc0, semsc1, after_prime):
    nb_total = cps // blk_b
    bufs = [(srcb0, dstb0), (srcb1, dstb1)]

    def wait_gather(buf, sem):
        pltpu.make_async_copy(h_hbm.at[srcb0.at[0]], buf, sem).wait()

    def wait_scatter(buf, sem):
        pltpu.make_async_copy(buf, acc.at[dstb0.at[0]], sem).wait()

    pltpu.sync_copy(src_hbm.at[pl.ds(base, blk_b)], srcb0)
    pltpu.sync_copy(dst_hbm.at[pl.ds(base, blk_b)], dstb0)
    pltpu.async_copy(h_hbm.at[srcb0.at[0]], rows0, semg0)
    pltpu.async_copy(h_hbm.at[srcb0.at[1]], rows1, semg1)
    after_prime()

    for nb in range(nb_total):
        srcb, dstb = bufs[nb % 2]
        nsrcb, ndstb = bufs[(nb + 1) % 2]
        have_next = nb + 1 < nb_total
        if have_next:
            nbase = base + (nb + 1) * blk_b
            ic_s = pltpu.async_copy(src_hbm.at[pl.ds(nbase, blk_b)],
                                    nsrcb, semi0)
            ic_d = pltpu.async_copy(dst_hbm.at[pl.ds(nbase, blk_b)],
                                    ndstb, semi1)

        @pl.loop(0, blk_b // 2 - 1)
        def _(p):
            j = 2 * p
            wait_gather(rows0, semg0)
            pltpu.sync_copy(rows0, acc.at[dstb.at[j]], add=True)
            pltpu.async_copy(h_hbm.at[srcb.at[j + 2]], rows0, semg0)
            wait_gather(rows1, semg1)
            pltpu.sync_copy(rows1, acc.at[dstb.at[j + 1]], add=True)
            pltpu.async_copy(h_hbm.at[srcb.at[j + 3]], rows1, semg1)

        wait_gather(rows0, semg0)
        pltpu.sync_copy(rows0, acc.at[dstb.at[blk_b - 2]], add=True)
        if have_next:
            ic_s.wait()
            ic_d.wait()
            pltpu.async_copy(h_hbm.at[nsrcb.at[0]], rows0, semg0)
        wait_gather(rows1, semg1)
        pltpu.sync_copy(rows1, acc.at[dstb.at[blk_b - 1]], add=True)
        if have_next:
            pltpu.async_copy(h_hbm.at[nsrcb.at[1]], rows1, semg1)


def _make_agg(split):
    fh = 128
    cps = CHUNKS // (NCORE * NSUB) if not split else CHUNKS // NSUB
    blk_b = 16
    out_t = (jax.ShapeDtypeStruct((N_EXT, fh), jnp.float32),
             jax.ShapeDtypeStruct((N_EXT, fh), jnp.float32))

    @functools.partial(
        pl.kernel, mesh=_sc_mesh(), out_type=out_t,
        scratch_types=[
            pltpu.VMEM((blk_b, CH), jnp.int32),
            pltpu.VMEM((blk_b, CH), jnp.int32),
            pltpu.VMEM((blk_b, CH), jnp.int32),
            pltpu.VMEM((blk_b, CH), jnp.int32),
            pltpu.VMEM((CH, fh), jnp.float32),
            pltpu.VMEM((CH, fh), jnp.float32),
            pltpu.VMEM_SHARED((N_EXT, fh), jnp.float32),
            pltpu.SemaphoreType.DMA,
            pltpu.SemaphoreType.DMA,
            pltpu.SemaphoreType.DMA,
            pltpu.SemaphoreType.DMA,
            pltpu.SemaphoreType.DMA,
            pltpu.SemaphoreType.DMA,
            pltpu.SemaphoreType.DMA,
        ],
    )
    def agg(ha_hbm, hb_hbm, src_hbm, dst_hbm, zeros_hbm, o0, o1,
            srcb0, dstb0, srcb1, dstb1, rows0, rows1, acc,
            semg0, semg1, semi0, semi1, semsc0, semsc1, semz):
        c = lax.axis_index("c")
        s = lax.axis_index("s")
        r0 = s * ROWS_PER_SUB
        zc = pltpu.async_copy(zeros_hbm.at[pl.ds(r0, ROWS_PER_SUB)],
                              acc.at[pl.ds(r0, ROWS_PER_SUB)], semz)

        def after_prime():
            zc.wait()
            plsc.subcore_barrier()

        if not split:
            base = (c * NSUB + s) * cps
        else:
            base = s * cps

        @pl.when(c == 0)
        def _():
            _chunk_loop(ha_hbm, src_hbm, dst_hbm, acc, base, cps, blk_b,
                        srcb0, dstb0, srcb1, dstb1, rows0, rows1,
                        semg0, semg1, semi0, semi1, semsc0, semsc1,
                        after_prime)

        @pl.when(c == 1)
        def _():
            _chunk_loop(hb_hbm, src_hbm, dst_hbm, acc, base, cps, blk_b,
                        srcb0, dstb0, srcb1, dstb1, rows0, rows1,
                        semg0, semg1, semi0, semi1, semsc0, semsc1,
                        after_prime)

        plsc.subcore_barrier()

        @pl.when(c == 0)
        def _():
            pltpu.sync_copy(acc.at[pl.ds(r0, ROWS_PER_SUB)],
                            o0.at[pl.ds(r0, ROWS_PER_SUB)])

        @pl.when(c == 1)
        def _():
            pltpu.sync_copy(acc.at[pl.ds(r0, ROWS_PER_SUB)],
                            o1.at[pl.ds(r0, ROWS_PER_SUB)])

    return agg


def _row_mask(i, y):
    rows = i * BLK + lax.broadcasted_iota(jnp.int32, (BLK, 1), 0)
    return jnp.where(rows < N, y, 0.0)


def _make_mlp1():
    def body(h, m0, m1, w, b, oa, ob):
        i = pl.program_id(0)
        x = h[...] + m0[...] + m1[...]
        y = jnp.dot(x, w[...], preferred_element_type=jnp.float32) + b[...]
        y = _row_mask(i, jnp.maximum(y, 0.0))
        oa[...] = y[:, :128]
        ob[...] = y[:, 128:]

    blk_in = pl.BlockSpec((BLK, 128), lambda i: (i, 0))
    return pl.pallas_call(
        body,
        grid=(N_EXT // BLK,),
        in_specs=[blk_in, blk_in, blk_in,
                  pl.BlockSpec((128, 256), lambda i: (0, 0)),
                  pl.BlockSpec((1, 256), lambda i: (0, 0))],
        out_specs=[pl.BlockSpec((BLK, 128), lambda i: (i, 0)),
                   pl.BlockSpec((BLK, 128), lambda i: (i, 0))],
        out_shape=(jax.ShapeDtypeStruct((N_EXT, 128), jnp.float32),
                   jax.ShapeDtypeStruct((N_EXT, 128), jnp.float32)),
    )


def _make_mlp_split(f_out, split_out):
    def body(ha, hb, ma, mb, w, b, *outs):
        i = pl.program_id(0)
        x = jnp.concatenate([ha[...] + ma[...], hb[...] + mb[...]], axis=1)
        y = jnp.dot(x, w[...], preferred_element_type=jnp.float32) + b[...]
        y = _row_mask(i, jnp.maximum(y, 0.0))
        if split_out:
            outs[0][...] = y[:, :f_out // 2]
            outs[1][...] = y[:, f_out // 2:]
        else:
            outs[0][...] = y

    blk_in = pl.BlockSpec((BLK, 128), lambda i: (i, 0))
    if split_out:
        out_specs = [pl.BlockSpec((BLK, f_out // 2), lambda i: (i, 0)),
                     pl.BlockSpec((BLK, f_out // 2), lambda i: (i, 0))]
        out_shape = (jax.ShapeDtypeStruct((N_EXT, f_out // 2), jnp.float32),
                     jax.ShapeDtypeStruct((N_EXT, f_out // 2), jnp.float32))
    else:
        out_specs = [pl.BlockSpec((BLK, f_out), lambda i: (i, 0))]
        out_shape = (jax.ShapeDtypeStruct((N_EXT, f_out), jnp.float32),)

    return pl.pallas_call(
        body,
        grid=(N_EXT // BLK,),
        in_specs=[blk_in, blk_in, blk_in, blk_in,
                  pl.BlockSpec((256, f_out), lambda i: (0, 0)),
                  pl.BlockSpec((1, f_out), lambda i: (0, 0))],
        out_specs=out_specs,
        out_shape=out_shape,
    )


def kernel(x, edge_index, W1, b1, W2, b2, W3, b3):
    src = edge_index[0].astype(jnp.int32)
    dst = edge_index[1].astype(jnp.int32)
    npad = E_PAD - E
    pad_ids = (jnp.arange(npad, dtype=jnp.int32) % (N_EXT - N)) + N
    srcm = jnp.concatenate([src, pad_ids]).reshape(CHUNKS, CH)
    dstm = jnp.concatenate([dst, pad_ids]).reshape(CHUNKS, CH)
    k = CHUNKS // NSUB
    srcm = srcm.reshape(k, NSUB, CH).transpose(1, 0, 2).reshape(CHUNKS, CH)
    dstm = dstm.reshape(k, NSUB, CH).transpose(1, 0, 2).reshape(CHUNKS, CH)
    h0 = jnp.zeros((N_EXT, 128), jnp.float32).at[:N].set(x)
    z128 = jnp.zeros((N_EXT, 128), jnp.float32)

    agg_full = _make_agg(split=False)
    agg_split = _make_agg(split=True)
    mlp1 = _make_mlp1()
    mlp2 = _make_mlp_split(256, split_out=True)
    mlp3 = _make_mlp_split(128, split_out=False)

    m0, m1 = agg_full(h0, h0, srcm, dstm, z128)
    ha, hb = mlp1(h0, m0, m1, W1, b1.reshape(1, -1))
    ma, mb = agg_split(ha, hb, srcm, dstm, z128)
    ha, hb = mlp2(ha, hb, ma, mb, W2, b2.reshape(1, -1))
    ma, mb = agg_split(ha, hb, srcm, dstm, z128)
    (h3,) = mlp3(ha, hb, ma, mb, W3, b3.reshape(1, -1))
    return h3[:N]


# speedup vs baseline: 1.3957x; 1.0399x over previous
import functools

import jax
import jax.numpy as jnp
from jax import lax
from jax.experimental import pallas as pl
from jax.experimental.pallas import tpu as pltpu
from jax.experimental.pallas import tpu_sc as plsc

N = 10000
E = 320000
N_EXT = 10240
CH = 128
NSUB = 16
NCORE = 2
CHUNKS = 2560
E_PAD = CHUNKS * CH
ROWS_PER_SUB = N_EXT // NSUB
BLK = 1024
BLK3 = 2000


def _sc_mesh():
    return plsc.VectorSubcoreMesh(core_axis_name="c", subcore_axis_name="s")


def _chunk_loop(h_hbm, src_hbm, dst_hbm, acc, base, cps, blk_b,
                srcb0, dstb0, srcb1, dstb1, rows0, rows1,
                semg0, semg1, semi0, semi1, semsc0, semsc1, after_prime):
    nb_total = cps // blk_b
    bufs = [(srcb0, dstb0), (srcb1, dstb1)]

    def wait_gather(buf, sem):
        pltpu.make_async_copy(h_hbm.at[srcb0.at[0]], buf, sem).wait()

    def wait_scatter(buf, sem):
        pltpu.make_async_copy(buf, acc.at[dstb0.at[0]], sem).wait()

    pltpu.sync_copy(src_hbm.at[pl.ds(base, blk_b)], srcb0)
    pltpu.sync_copy(dst_hbm.at[pl.ds(base, blk_b)], dstb0)
    pltpu.async_copy(h_hbm.at[srcb0.at[0]], rows0, semg0)
    pltpu.async_copy(h_hbm.at[srcb0.at[1]], rows1, semg1)
    after_prime()

    for nb in range(nb_total):
        srcb, dstb = bufs[nb % 2]
        nsrcb, ndstb = bufs[(nb + 1) % 2]
        have_next = nb + 1 < nb_total
        if have_next:
            nbase = base + (nb + 1) * blk_b
            ic_s = pltpu.async_copy(src_hbm.at[pl.ds(nbase, blk_b)],
                                    nsrcb, semi0)
            ic_d = pltpu.async_copy(dst_hbm.at[pl.ds(nbase, blk_b)],
                                    ndstb, semi1)

        @pl.loop(0, blk_b // 2 - 1)
        def _(p):
            j = 2 * p
            wait_gather(rows0, semg0)
            pltpu.sync_copy(rows0, acc.at[dstb.at[j]], add=True)
            pltpu.async_copy(h_hbm.at[srcb.at[j + 2]], rows0, semg0)
            wait_gather(rows1, semg1)
            pltpu.sync_copy(rows1, acc.at[dstb.at[j + 1]], add=True)
            pltpu.async_copy(h_hbm.at[srcb.at[j + 3]], rows1, semg1)

        wait_gather(rows0, semg0)
        pltpu.sync_copy(rows0, acc.at[dstb.at[blk_b - 2]], add=True)
        if have_next:
            ic_s.wait()
            ic_d.wait()
            pltpu.async_copy(h_hbm.at[nsrcb.at[0]], rows0, semg0)
        wait_gather(rows1, semg1)
        pltpu.sync_copy(rows1, acc.at[dstb.at[blk_b - 1]], add=True)
        if have_next:
            pltpu.async_copy(h_hbm.at[nsrcb.at[1]], rows1, semg1)


def _make_agg(split):
    fh = 128
    cps = CHUNKS // (NCORE * NSUB) if not split else CHUNKS // NSUB
    blk_b = 16
    out_t = (jax.ShapeDtypeStruct((N_EXT, fh), jnp.float32),
             jax.ShapeDtypeStruct((N_EXT, fh), jnp.float32))

    @functools.partial(
        pl.kernel, mesh=_sc_mesh(), out_type=out_t,
        scratch_types=[
            pltpu.VMEM((blk_b, CH), jnp.int32),
            pltpu.VMEM((blk_b, CH), jnp.int32),
            pltpu.VMEM((blk_b, CH), jnp.int32),
            pltpu.VMEM((blk_b, CH), jnp.int32),
            pltpu.VMEM((CH, fh), jnp.float32),
            pltpu.VMEM((CH, fh), jnp.float32),
            pltpu.VMEM_SHARED((N_EXT, fh), jnp.float32),
            pltpu.SemaphoreType.DMA,
            pltpu.SemaphoreType.DMA,
            pltpu.SemaphoreType.DMA,
            pltpu.SemaphoreType.DMA,
            pltpu.SemaphoreType.DMA,
            pltpu.SemaphoreType.DMA,
            pltpu.SemaphoreType.DMA,
        ],
    )
    def agg(ha_hbm, hb_hbm, src_hbm, dst_hbm, zeros_hbm, o0, o1,
            srcb0, dstb0, srcb1, dstb1, rows0, rows1, acc,
            semg0, semg1, semi0, semi1, semsc0, semsc1, semz):
        c = lax.axis_index("c")
        s = lax.axis_index("s")
        r0 = s * ROWS_PER_SUB
        zc = pltpu.async_copy(zeros_hbm.at[pl.ds(r0, ROWS_PER_SUB)],
                              acc.at[pl.ds(r0, ROWS_PER_SUB)], semz)

        def after_prime():
            zc.wait()
            plsc.subcore_barrier()

        if not split:
            base = (c * NSUB + s) * cps
        else:
            base = s * cps

        @pl.when(c == 0)
        def _():
            _chunk_loop(ha_hbm, src_hbm, dst_hbm, acc, base, cps, blk_b,
                        srcb0, dstb0, srcb1, dstb1, rows0, rows1,
                        semg0, semg1, semi0, semi1, semsc0, semsc1,
                        after_prime)

        @pl.when(c == 1)
        def _():
            _chunk_loop(hb_hbm, src_hbm, dst_hbm, acc, base, cps, blk_b,
                        srcb0, dstb0, srcb1, dstb1, rows0, rows1,
                        semg0, semg1, semi0, semi1, semsc0, semsc1,
                        after_prime)

        plsc.subcore_barrier()

        @pl.when(c == 0)
        def _():
            pltpu.sync_copy(acc.at[pl.ds(r0, ROWS_PER_SUB)],
                            o0.at[pl.ds(r0, ROWS_PER_SUB)])

        @pl.when(c == 1)
        def _():
            pltpu.sync_copy(acc.at[pl.ds(r0, ROWS_PER_SUB)],
                            o1.at[pl.ds(r0, ROWS_PER_SUB)])

    return agg


def _row_mask(i, y, blk):
    rows = i * blk + lax.broadcasted_iota(jnp.int32, (blk, 1), 0)
    return jnp.where(rows < N, y, 0.0)


def _make_mlp1():
    def body(h, m0, m1, w, b, oa, ob):
        i = pl.program_id(0)
        x = h[...] + m0[...] + m1[...]
        y = jnp.dot(x, w[...], preferred_element_type=jnp.float32) + b[...]
        y = _row_mask(i, jnp.maximum(y, 0.0), BLK)
        oa[...] = y[:, :128]
        ob[...] = y[:, 128:]

    blk_in = pl.BlockSpec((BLK, 128), lambda i: (i, 0))
    return pl.pallas_call(
        body,
        grid=(N_EXT // BLK,),
        in_specs=[blk_in, blk_in, blk_in,
                  pl.BlockSpec((128, 256), lambda i: (0, 0)),
                  pl.BlockSpec((1, 256), lambda i: (0, 0))],
        out_specs=[pl.BlockSpec((BLK, 128), lambda i: (i, 0)),
                   pl.BlockSpec((BLK, 128), lambda i: (i, 0))],
        out_shape=(jax.ShapeDtypeStruct((N_EXT, 128), jnp.float32),
                   jax.ShapeDtypeStruct((N_EXT, 128), jnp.float32)),
    )


def _make_mlp_split(f_out, split_out):
    blk = BLK if split_out else BLK3

    def body(ha, hb, ma, mb, w, b, *outs):
        i = pl.program_id(0)
        x = jnp.concatenate([ha[...] + ma[...], hb[...] + mb[...]], axis=1)
        y = jnp.dot(x, w[...], preferred_element_type=jnp.float32) + b[...]
        y = jnp.maximum(y, 0.0)
        if split_out:
            y = _row_mask(i, y, blk)
            outs[0][...] = y[:, :f_out // 2]
            outs[1][...] = y[:, f_out // 2:]
        else:
            outs[0][...] = y

    blk_in = pl.BlockSpec((blk, 128), lambda i: (i, 0))
    if split_out:
        grid = (N_EXT // blk,)
        out_specs = [pl.BlockSpec((blk, f_out // 2), lambda i: (i, 0)),
                     pl.BlockSpec((blk, f_out // 2), lambda i: (i, 0))]
        out_shape = (jax.ShapeDtypeStruct((N_EXT, f_out // 2), jnp.float32),
                     jax.ShapeDtypeStruct((N_EXT, f_out // 2), jnp.float32))
    else:
        grid = (N // blk,)
        out_specs = [pl.BlockSpec((blk, f_out), lambda i: (i, 0))]
        out_shape = (jax.ShapeDtypeStruct((N, f_out), jnp.float32),)

    return pl.pallas_call(
        body,
        grid=grid,
        in_specs=[blk_in, blk_in, blk_in, blk_in,
                  pl.BlockSpec((256, f_out), lambda i: (0, 0)),
                  pl.BlockSpec((1, f_out), lambda i: (0, 0))],
        out_specs=out_specs,
        out_shape=out_shape,
    )


def kernel(x, edge_index, W1, b1, W2, b2, W3, b3):
    src = edge_index[0].astype(jnp.int32)
    dst = edge_index[1].astype(jnp.int32)
    npad = E_PAD - E
    pad_ids = (jnp.arange(npad, dtype=jnp.int32) % (N_EXT - N)) + N
    srcm = jnp.concatenate([src, pad_ids]).reshape(CHUNKS, CH)
    dstm = jnp.concatenate([dst, pad_ids]).reshape(CHUNKS, CH)
    k = CHUNKS // NSUB
    srcm = srcm.reshape(k, NSUB, CH).transpose(1, 0, 2).reshape(CHUNKS, CH)
    dstm = dstm.reshape(k, NSUB, CH).transpose(1, 0, 2).reshape(CHUNKS, CH)
    h0 = jnp.zeros((N_EXT, 128), jnp.float32).at[:N].set(x)
    z128 = jnp.zeros((N_EXT, 128), jnp.float32)

    agg_full = _make_agg(split=False)
    agg_split = _make_agg(split=True)
    mlp1 = _make_mlp1()
    mlp2 = _make_mlp_split(256, split_out=True)
    mlp3 = _make_mlp_split(128, split_out=False)

    m0, m1 = agg_full(h0, h0, srcm, dstm, z128)
    ha, hb = mlp1(h0, m0, m1, W1, b1.reshape(1, -1))
    ma, mb = agg_split(ha, hb, srcm, dstm, z128)
    ha, hb = mlp2(ha, hb, ma, mb, W2, b2.reshape(1, -1))
    ma, mb = agg_split(ha, hb, srcm, dstm, z128)
    (h3,) = mlp3(ha, hb, ma, mb, W3, b3.reshape(1, -1))
    return h3


# speedup vs baseline: 1.3957x; 1.0000x over previous
import functools

import jax
import jax.numpy as jnp
from jax import lax
from jax.experimental import pallas as pl
from jax.experimental.pallas import tpu as pltpu
from jax.experimental.pallas import tpu_sc as plsc

N = 10000
E = 320000
N_EXT = 10240
CH = 128
NSUB = 16
NCORE = 2
CHUNKS = 2560
E_PAD = CHUNKS * CH
ROWS_PER_SUB = N_EXT // NSUB
BLK = 1024
BLK3 = 2000


def _sc_mesh():
    return plsc.VectorSubcoreMesh(core_axis_name="c", subcore_axis_name="s")


def _chunk_loop(h_hbm, src_hbm, dst_hbm, acc, base, cps, blk_b,
                srcb0, dstb0, srcb1, dstb1, rows0, rows1,
                semg0, semg1, semi0, semi1, after_prime):
    nb_total = cps // blk_b
    bufs = [(srcb0, dstb0), (srcb1, dstb1)]

    def wait_gather(buf, sem):
        pltpu.make_async_copy(h_hbm.at[srcb0.at[0]], buf, sem).wait()

    pltpu.sync_copy(src_hbm.at[pl.ds(base, blk_b)], srcb0)
    pltpu.sync_copy(dst_hbm.at[pl.ds(base, blk_b)], dstb0)
    pltpu.async_copy(h_hbm.at[srcb0.at[0]], rows0, semg0)
    pltpu.async_copy(h_hbm.at[srcb0.at[1]], rows1, semg1)
    after_prime()

    for nb in range(nb_total):
        srcb, dstb = bufs[nb % 2]
        nsrcb, ndstb = bufs[(nb + 1) % 2]
        have_next = nb + 1 < nb_total
        if have_next:
            nbase = base + (nb + 1) * blk_b
            ic_s = pltpu.async_copy(src_hbm.at[pl.ds(nbase, blk_b)],
                                    nsrcb, semi0)
            ic_d = pltpu.async_copy(dst_hbm.at[pl.ds(nbase, blk_b)],
                                    ndstb, semi1)

        @pl.loop(0, blk_b // 2 - 1)
        def _(p):
            j = 2 * p
            wait_gather(rows0, semg0)
            pltpu.sync_copy(rows0, acc.at[dstb.at[j]], add=True)
            pltpu.async_copy(h_hbm.at[srcb.at[j + 2]], rows0, semg0)
            wait_gather(rows1, semg1)
            pltpu.sync_copy(rows1, acc.at[dstb.at[j + 1]], add=True)
            pltpu.async_copy(h_hbm.at[srcb.at[j + 3]], rows1, semg1)

        wait_gather(rows0, semg0)
        pltpu.sync_copy(rows0, acc.at[dstb.at[blk_b - 2]], add=True)
        if have_next:
            ic_s.wait()
            ic_d.wait()
            pltpu.async_copy(h_hbm.at[nsrcb.at[0]], rows0, semg0)
        wait_gather(rows1, semg1)
        pltpu.sync_copy(rows1, acc.at[dstb.at[blk_b - 1]], add=True)
        if have_next:
            pltpu.async_copy(h_hbm.at[nsrcb.at[1]], rows1, semg1)


def _make_agg(split):
    fh = 128
    cps = CHUNKS // (NCORE * NSUB) if not split else CHUNKS // NSUB
    blk_b = 16
    out_t = (jax.ShapeDtypeStruct((N_EXT, fh), jnp.float32),
             jax.ShapeDtypeStruct((N_EXT, fh), jnp.float32))

    @functools.partial(
        pl.kernel, mesh=_sc_mesh(), out_type=out_t,
        scratch_types=[
            pltpu.VMEM((blk_b, CH), jnp.int32),
            pltpu.VMEM((blk_b, CH), jnp.int32),
            pltpu.VMEM((blk_b, CH), jnp.int32),
            pltpu.VMEM((blk_b, CH), jnp.int32),
            pltpu.VMEM((CH, fh), jnp.float32),
            pltpu.VMEM((CH, fh), jnp.float32),
            pltpu.VMEM_SHARED((N_EXT, fh), jnp.float32),
            pltpu.SemaphoreType.DMA,
            pltpu.SemaphoreType.DMA,
            pltpu.SemaphoreType.DMA,
            pltpu.SemaphoreType.DMA,
            pltpu.SemaphoreType.DMA,
        ],
    )
    def agg(ha_hbm, hb_hbm, src_hbm, dst_hbm, zeros_hbm, o0, o1,
            srcb0, dstb0, srcb1, dstb1, rows0, rows1, acc,
            semg0, semg1, semi0, semi1, semz):
        c = lax.axis_index("c")
        s = lax.axis_index("s")
        r0 = s * ROWS_PER_SUB
        zc = pltpu.async_copy(zeros_hbm.at[pl.ds(r0, ROWS_PER_SUB)],
                              acc.at[pl.ds(r0, ROWS_PER_SUB)], semz)

        def after_prime():
            zc.wait()
            plsc.subcore_barrier()

        if not split:
            base = (c * NSUB + s) * cps
        else:
            base = s * cps

        @pl.when(c == 0)
        def _():
            _chunk_loop(ha_hbm, src_hbm, dst_hbm, acc, base, cps, blk_b,
                        srcb0, dstb0, srcb1, dstb1, rows0, rows1,
                        semg0, semg1, semi0, semi1, after_prime)

        @pl.when(c == 1)
        def _():
            _chunk_loop(hb_hbm, src_hbm, dst_hbm, acc, base, cps, blk_b,
                        srcb0, dstb0, srcb1, dstb1, rows0, rows1,
                        semg0, semg1, semi0, semi1, after_prime)

        plsc.subcore_barrier()

        @pl.when(c == 0)
        def _():
            pltpu.sync_copy(acc.at[pl.ds(r0, ROWS_PER_SUB)],
                            o0.at[pl.ds(r0, ROWS_PER_SUB)])

        @pl.when(c == 1)
        def _():
            pltpu.sync_copy(acc.at[pl.ds(r0, ROWS_PER_SUB)],
                            o1.at[pl.ds(r0, ROWS_PER_SUB)])

    return agg


def _row_mask(i, y, blk):
    rows = i * blk + lax.broadcasted_iota(jnp.int32, (blk, 1), 0)
    return jnp.where(rows < N, y, 0.0)


def _make_mlp1():
    def body(h, m0, m1, w, b, oa, ob):
        i = pl.program_id(0)
        x = h[...] + m0[...] + m1[...]
        y = jnp.dot(x, w[...], preferred_element_type=jnp.float32) + b[...]
        y = _row_mask(i, jnp.maximum(y, 0.0), BLK)
        oa[...] = y[:, :128]
        ob[...] = y[:, 128:]

    blk_in = pl.BlockSpec((BLK, 128), lambda i: (i, 0))
    return pl.pallas_call(
        body,
        grid=(N_EXT // BLK,),
        in_specs=[blk_in, blk_in, blk_in,
                  pl.BlockSpec((128, 256), lambda i: (0, 0)),
                  pl.BlockSpec((1, 256), lambda i: (0, 0))],
        out_specs=[pl.BlockSpec((BLK, 128), lambda i: (i, 0)),
                   pl.BlockSpec((BLK, 128), lambda i: (i, 0))],
        out_shape=(jax.ShapeDtypeStruct((N_EXT, 128), jnp.float32),
                   jax.ShapeDtypeStruct((N_EXT, 128), jnp.float32)),
    )


def _make_mlp_split(f_out, split_out):
    blk = BLK if split_out else BLK3

    def body(ha, hb, ma, mb, w, b, *outs):
        i = pl.program_id(0)
        x = jnp.concatenate([ha[...] + ma[...], hb[...] + mb[...]], axis=1)
        y = jnp.dot(x, w[...], preferred_element_type=jnp.float32) + b[...]
        y = jnp.maximum(y, 0.0)
        if split_out:
            y = _row_mask(i, y, blk)
            outs[0][...] = y[:, :f_out // 2]
            outs[1][...] = y[:, f_out // 2:]
        else:
            outs[0][...] = y

    blk_in = pl.BlockSpec((blk, 128), lambda i: (i, 0))
    if split_out:
        grid = (N_EXT // blk,)
        out_specs = [pl.BlockSpec((blk, f_out // 2), lambda i: (i, 0)),
                     pl.BlockSpec((blk, f_out // 2), lambda i: (i, 0))]
        out_shape = (jax.ShapeDtypeStruct((N_EXT, f_out // 2), jnp.float32),
                     jax.ShapeDtypeStruct((N_EXT, f_out // 2), jnp.float32))
    else:
        grid = (N // blk,)
        out_specs = [pl.BlockSpec((blk, f_out), lambda i: (i, 0))]
        out_shape = (jax.ShapeDtypeStruct((N, f_out), jnp.float32),)

    return pl.pallas_call(
        body,
        grid=grid,
        in_specs=[blk_in, blk_in, blk_in, blk_in,
                  pl.BlockSpec((256, f_out), lambda i: (0, 0)),
                  pl.BlockSpec((1, f_out), lambda i: (0, 0))],
        out_specs=out_specs,
        out_shape=out_shape,
    )


def kernel(x, edge_index, W1, b1, W2, b2, W3, b3):
    src = edge_index[0].astype(jnp.int32)
    dst = edge_index[1].astype(jnp.int32)
    npad = E_PAD - E
    pad_ids = (jnp.arange(npad, dtype=jnp.int32) % (N_EXT - N)) + N
    srcm = jnp.concatenate([src, pad_ids]).reshape(CHUNKS, CH)
    dstm = jnp.concatenate([dst, pad_ids]).reshape(CHUNKS, CH)
    k = CHUNKS // NSUB
    srcm = srcm.reshape(k, NSUB, CH).transpose(1, 0, 2).reshape(CHUNKS, CH)
    dstm = dstm.reshape(k, NSUB, CH).transpose(1, 0, 2).reshape(CHUNKS, CH)
    h0 = jnp.zeros((N_EXT, 128), jnp.float32).at[:N].set(x)
    z128 = jnp.zeros((N_EXT, 128), jnp.float32)

    agg_full = _make_agg(split=False)
    agg_split = _make_agg(split=True)
    mlp1 = _make_mlp1()
    mlp2 = _make_mlp_split(256, split_out=True)
    mlp3 = _make_mlp_split(128, split_out=False)

    m0, m1 = agg_full(h0, h0, srcm, dstm, z128)
    ha, hb = mlp1(h0, m0, m1, W1, b1.reshape(1, -1))
    ma, mb = agg_split(ha, hb, srcm, dstm, z128)
    ha, hb = mlp2(ha, hb, ma, mb, W2, b2.reshape(1, -1))
    ma, mb = agg_split(ha, hb, srcm, dstm, z128)
    (h3,) = mlp3(ha, hb, ma, mb, W3, b3.reshape(1, -1))
    return h3


# speedup vs baseline: 1.4029x; 1.0052x over previous
import functools

import jax
import jax.numpy as jnp
from jax import lax
from jax.experimental import pallas as pl
from jax.experimental.pallas import tpu as pltpu
from jax.experimental.pallas import tpu_sc as plsc

N = 10000
E = 320000
N_EXT = 10240
CH = 128
NSUB = 16
NCORE = 2
CHUNKS = 2560
E_PAD = CHUNKS * CH
ROWS_PER_SUB = N_EXT // NSUB
BLK = 2048
BLK3 = 2000


def _sc_mesh():
    return plsc.VectorSubcoreMesh(core_axis_name="c", subcore_axis_name="s")


def _chunk_loop(h_hbm, src_hbm, dst_hbm, acc, base, cps, blk_b,
                srcb0, dstb0, srcb1, dstb1, rows0, rows1,
                semg0, semg1, semi0, semi1, after_prime):
    nb_total = cps // blk_b
    bufs = [(srcb0, dstb0), (srcb1, dstb1)]

    def wait_gather(buf, sem):
        pltpu.make_async_copy(h_hbm.at[srcb0.at[0]], buf, sem).wait()

    pltpu.sync_copy(src_hbm.at[pl.ds(base, blk_b)], srcb0)
    pltpu.sync_copy(dst_hbm.at[pl.ds(base, blk_b)], dstb0)
    pltpu.async_copy(h_hbm.at[srcb0.at[0]], rows0, semg0)
    pltpu.async_copy(h_hbm.at[srcb0.at[1]], rows1, semg1)
    after_prime()

    for nb in range(nb_total):
        srcb, dstb = bufs[nb % 2]
        nsrcb, ndstb = bufs[(nb + 1) % 2]
        have_next = nb + 1 < nb_total
        if have_next:
            nbase = base + (nb + 1) * blk_b
            ic_s = pltpu.async_copy(src_hbm.at[pl.ds(nbase, blk_b)],
                                    nsrcb, semi0)
            ic_d = pltpu.async_copy(dst_hbm.at[pl.ds(nbase, blk_b)],
                                    ndstb, semi1)

        @pl.loop(0, blk_b // 2 - 1)
        def _(p):
            j = 2 * p
            wait_gather(rows0, semg0)
            pltpu.sync_copy(rows0, acc.at[dstb.at[j]], add=True)
            pltpu.async_copy(h_hbm.at[srcb.at[j + 2]], rows0, semg0)
            wait_gather(rows1, semg1)
            pltpu.sync_copy(rows1, acc.at[dstb.at[j + 1]], add=True)
            pltpu.async_copy(h_hbm.at[srcb.at[j + 3]], rows1, semg1)

        wait_gather(rows0, semg0)
        pltpu.sync_copy(rows0, acc.at[dstb.at[blk_b - 2]], add=True)
        if have_next:
            ic_s.wait()
            ic_d.wait()
            pltpu.async_copy(h_hbm.at[nsrcb.at[0]], rows0, semg0)
        wait_gather(rows1, semg1)
        pltpu.sync_copy(rows1, acc.at[dstb.at[blk_b - 1]], add=True)
        if have_next:
            pltpu.async_copy(h_hbm.at[nsrcb.at[1]], rows1, semg1)


def _make_agg(split):
    fh = 128
    cps = CHUNKS // (NCORE * NSUB) if not split else CHUNKS // NSUB
    blk_b = 16
    out_t = (jax.ShapeDtypeStruct((N_EXT, fh), jnp.float32),
             jax.ShapeDtypeStruct((N_EXT, fh), jnp.float32))

    @functools.partial(
        pl.kernel, mesh=_sc_mesh(), out_type=out_t,
        scratch_types=[
            pltpu.VMEM((blk_b, CH), jnp.int32),
            pltpu.VMEM((blk_b, CH), jnp.int32),
            pltpu.VMEM((blk_b, CH), jnp.int32),
            pltpu.VMEM((blk_b, CH), jnp.int32),
            pltpu.VMEM((CH, fh), jnp.float32),
            pltpu.VMEM((CH, fh), jnp.float32),
            pltpu.VMEM_SHARED((N_EXT, fh), jnp.float32),
            pltpu.SemaphoreType.DMA,
            pltpu.SemaphoreType.DMA,
            pltpu.SemaphoreType.DMA,
            pltpu.SemaphoreType.DMA,
            pltpu.SemaphoreType.DMA,
        ],
    )
    def agg(ha_hbm, hb_hbm, src_hbm, dst_hbm, zeros_hbm, o0, o1,
            srcb0, dstb0, srcb1, dstb1, rows0, rows1, acc,
            semg0, semg1, semi0, semi1, semz):
        c = lax.axis_index("c")
        s = lax.axis_index("s")
        r0 = s * ROWS_PER_SUB
        zc = pltpu.async_copy(zeros_hbm.at[pl.ds(r0, ROWS_PER_SUB)],
                              acc.at[pl.ds(r0, ROWS_PER_SUB)], semz)

        def after_prime():
            zc.wait()
            plsc.subcore_barrier()

        if not split:
            base = (c * NSUB + s) * cps
        else:
            base = s * cps

        @pl.when(c == 0)
        def _():
            _chunk_loop(ha_hbm, src_hbm, dst_hbm, acc, base, cps, blk_b,
                        srcb0, dstb0, srcb1, dstb1, rows0, rows1,
                        semg0, semg1, semi0, semi1, after_prime)

        @pl.when(c == 1)
        def _():
            _chunk_loop(hb_hbm, src_hbm, dst_hbm, acc, base, cps, blk_b,
                        srcb0, dstb0, srcb1, dstb1, rows0, rows1,
                        semg0, semg1, semi0, semi1, after_prime)

        plsc.subcore_barrier()

        @pl.when(c == 0)
        def _():
            pltpu.sync_copy(acc.at[pl.ds(r0, ROWS_PER_SUB)],
                            o0.at[pl.ds(r0, ROWS_PER_SUB)])

        @pl.when(c == 1)
        def _():
            pltpu.sync_copy(acc.at[pl.ds(r0, ROWS_PER_SUB)],
                            o1.at[pl.ds(r0, ROWS_PER_SUB)])

    return agg


def _row_mask(i, y, blk):
    rows = i * blk + lax.broadcasted_iota(jnp.int32, (blk, 1), 0)
    return jnp.where(rows < N, y, 0.0)


def _make_mlp1():
    def body(h, m0, m1, w, b, oa, ob):
        i = pl.program_id(0)
        x = h[...] + m0[...] + m1[...]
        y = jnp.dot(x, w[...], preferred_element_type=jnp.float32) + b[...]
        y = _row_mask(i, jnp.maximum(y, 0.0), BLK)
        oa[...] = y[:, :128]
        ob[...] = y[:, 128:]

    blk_in = pl.BlockSpec((BLK, 128), lambda i: (i, 0))
    return pl.pallas_call(
        body,
        grid=(N_EXT // BLK,),
        in_specs=[blk_in, blk_in, blk_in,
                  pl.BlockSpec((128, 256), lambda i: (0, 0)),
                  pl.BlockSpec((1, 256), lambda i: (0, 0))],
        out_specs=[pl.BlockSpec((BLK, 128), lambda i: (i, 0)),
                   pl.BlockSpec((BLK, 128), lambda i: (i, 0))],
        out_shape=(jax.ShapeDtypeStruct((N_EXT, 128), jnp.float32),
                   jax.ShapeDtypeStruct((N_EXT, 128), jnp.float32)),
    )


def _make_mlp_split(f_out, split_out):
    blk = BLK if split_out else BLK3

    def body(ha, hb, ma, mb, w, b, *outs):
        i = pl.program_id(0)
        x = jnp.concatenate([ha[...] + ma[...], hb[...] + mb[...]], axis=1)
        y = jnp.dot(x, w[...], preferred_element_type=jnp.float32) + b[...]
        y = jnp.maximum(y, 0.0)
        if split_out:
            y = _row_mask(i, y, blk)
            outs[0][...] = y[:, :f_out // 2]
            outs[1][...] = y[:, f_out // 2:]
        else:
            outs[0][...] = y

    blk_in = pl.BlockSpec((blk, 128), lambda i: (i, 0))
    if split_out:
        grid = (N_EXT // blk,)
        out_specs = [pl.BlockSpec((blk, f_out // 2), lambda i: (i, 0)),
                     pl.BlockSpec((blk, f_out // 2), lambda i: (i, 0))]
        out_shape = (jax.ShapeDtypeStruct((N_EXT, f_out // 2), jnp.float32),
                     jax.ShapeDtypeStruct((N_EXT, f_out // 2), jnp.float32))
    else:
        grid = (N // blk,)
        out_specs = [pl.BlockSpec((blk, f_out), lambda i: (i, 0))]
        out_shape = (jax.ShapeDtypeStruct((N, f_out), jnp.float32),)

    return pl.pallas_call(
        body,
        grid=grid,
        in_specs=[blk_in, blk_in, blk_in, blk_in,
                  pl.BlockSpec((256, f_out), lambda i: (0, 0)),
                  pl.BlockSpec((1, f_out), lambda i: (0, 0))],
        out_specs=out_specs,
        out_shape=out_shape,
    )


def kernel(x, edge_index, W1, b1, W2, b2, W3, b3):
    src = edge_index[0].astype(jnp.int32)
    dst = edge_index[1].astype(jnp.int32)
    npad = E_PAD - E
    pad_ids = (jnp.arange(npad, dtype=jnp.int32) % (N_EXT - N)) + N
    srcm = jnp.concatenate([src, pad_ids]).reshape(CHUNKS, CH)
    dstm = jnp.concatenate([dst, pad_ids]).reshape(CHUNKS, CH)
    k = CHUNKS // NSUB
    srcm = srcm.reshape(k, NSUB, CH).transpose(1, 0, 2).reshape(CHUNKS, CH)
    dstm = dstm.reshape(k, NSUB, CH).transpose(1, 0, 2).reshape(CHUNKS, CH)
    h0 = jnp.zeros((N_EXT, 128), jnp.float32).at[:N].set(x)
    z128 = jnp.zeros((N_EXT, 128), jnp.float32)

    agg_full = _make_agg(split=False)
    agg_split = _make_agg(split=True)
    mlp1 = _make_mlp1()
    mlp2 = _make_mlp_split(256, split_out=True)
    mlp3 = _make_mlp_split(128, split_out=False)

    m0, m1 = agg_full(h0, h0, srcm, dstm, z128)
    ha, hb = mlp1(h0, m0, m1, W1, b1.reshape(1, -1))
    ma, mb = agg_split(ha, hb, srcm, dstm, z128)
    ha, hb = mlp2(ha, hb, ma, mb, W2, b2.reshape(1, -1))
    ma, mb = agg_split(ha, hb, srcm, dstm, z128)
    (h3,) = mlp3(ha, hb, ma, mb, W3, b3.reshape(1, -1))
    return h3
